# Initial kernel scaffold; baseline (speedup 1.0000x reference)
import functools

import jax
import jax.numpy as jnp
from jax import lax
from jax.experimental import pallas as pl
from jax.experimental.pallas import tpu as pltpu
from jax.experimental.pallas import tpu_sc as plsc

N = 10000
E = 320000
D = 128
CHUNK = 128
NUM_CHUNKS = E // CHUNK
NT = 16
BASE_CHUNKS = NUM_CHUNKS // NT
EXTRA = NUM_CHUNKS - BASE_CHUNKS * NT
DEGW = 16
ZROWS = N // 10


def _build_sc_aggregate():
    mesh = plsc.VectorSubcoreMesh(core_axis_name="c", subcore_axis_name="s")

    @functools.partial(
        pl.kernel,
        out_type=[
            jax.ShapeDtypeStruct((2, N, D), jnp.float32),
            jax.ShapeDtypeStruct((2, N, DEGW), jnp.float32),
        ],
        mesh=mesh,
        scratch_types=[
            pltpu.VMEM((CHUNK,), jnp.int32),
            pltpu.VMEM((1, CHUNK), jnp.int32),
            pltpu.VMEM((CHUNK, D), jnp.float32),
            pltpu.VMEM((CHUNK, DEGW), jnp.float32),
            pltpu.VMEM_SHARED((N, D), jnp.float32),
            pltpu.VMEM_SHARED((N, DEGW), jnp.float32),
            pltpu.SemaphoreType.DMA,
        ],
    )
    def sc_agg(x_hbm, edge_hbm, z128_hbm, z16_hbm, ones_hbm,
               sums_out, deg_out,
               idx_g, idx_s, rows, ones_v, acc, dacc, sem):
        cid = lax.axis_index("c")
        tid = lax.axis_index("s")
        gdim = cid
        sdim = 1 - cid

        @pl.when(tid < 10)
        def _():
            r0 = tid * ZROWS
            pltpu.sync_copy(z128_hbm.at[pl.ds(r0, ZROWS)], acc.at[pl.ds(r0, ZROWS)])
            pltpu.sync_copy(z16_hbm.at[pl.ds(r0, ZROWS)], dacc.at[pl.ds(r0, ZROWS)])
        pltpu.sync_copy(ones_hbm, ones_v)
        plsc.subcore_barrier()

        nch = BASE_CHUNKS + jnp.where(tid < EXTRA, 1, 0)
        cbase = tid * BASE_CHUNKS + jnp.minimum(tid, EXTRA)

        def body(c, carry):
            off = (cbase + c) * CHUNK
            pltpu.sync_copy(edge_hbm.at[gdim, pl.ds(off, CHUNK)], idx_g)
            pltpu.sync_copy(edge_hbm.at[sdim, pl.ds(off, CHUNK)], idx_s.at[0])
            pltpu.async_copy(x_hbm.at[idx_g], rows, sem).wait()
            pltpu.sync_copy(rows, acc.at[idx_s.at[0]], add=True)
            pltpu.sync_copy(ones_v, dacc.at[idx_s.at[0]], add=True)
            return carry

        lax.fori_loop(0, nch, body, 0)
        plsc.subcore_barrier()

        @pl.when(tid < 10)
        def _():
            r0 = tid * ZROWS
            pltpu.sync_copy(acc.at[pl.ds(r0, ZROWS)], sums_out.at[cid, pl.ds(r0, ZROWS)])
            pltpu.sync_copy(dacc.at[pl.ds(r0, ZROWS)], deg_out.at[cid, pl.ds(r0, ZROWS)])

    return sc_agg


_SC_AGG = _build_sc_aggregate()

_TC_ROWS = 1000


def _tc_body(x_ref, s_ref, d_ref, ws_ref, w2_ref, w3_ref, b_ref, o_ref):
    xb = x_ref[...]
    s2d = s_ref[0]
    d2s = s_ref[1]
    r_in = 0.5 / jnp.maximum(d_ref[0, :, 0:1], 1.0)
    r_out = 0.5 / jnp.maximum(d_ref[1, :, 0:1], 1.0)
    acc = jnp.dot(xb, ws_ref[...], preferred_element_type=jnp.float32)
    acc = acc + r_in * jnp.dot(s2d, w2_ref[...], preferred_element_type=jnp.float32)
    acc = acc + r_out * jnp.dot(d2s, w3_ref[...], preferred_element_type=jnp.float32)
    o_ref[...] = acc + b_ref[...]


def _tc_combine(x, sums, degs, ws_t, w2_t, w3_t, b_all):
    return pl.pallas_call(
        _tc_body,
        grid=(N // _TC_ROWS,),
        in_specs=[
            pl.BlockSpec((_TC_ROWS, D), lambda i: (i, 0)),
            pl.BlockSpec((2, _TC_ROWS, D), lambda i: (0, i, 0)),
            pl.BlockSpec((2, _TC_ROWS, DEGW), lambda i: (0, i, 0)),
            pl.BlockSpec((D, D), lambda i: (0, 0)),
            pl.BlockSpec((D, D), lambda i: (0, 0)),
            pl.BlockSpec((D, D), lambda i: (0, 0)),
            pl.BlockSpec((1, D), lambda i: (0, 0)),
        ],
        out_specs=pl.BlockSpec((_TC_ROWS, D), lambda i: (i, 0)),
        out_shape=jax.ShapeDtypeStruct((N, D), jnp.float32),
    )(x, sums, degs, ws_t, w2_t, w3_t, b_all)


def kernel(x, edge_index, W_self, b_self, W_s2d, b_s2d, W_d2s, b_d2s):
    z128 = jnp.zeros((N, D), jnp.float32)
    z16 = jnp.zeros((N, DEGW), jnp.float32)
    ones = jnp.ones((CHUNK, DEGW), jnp.float32)
    sums, degs = _SC_AGG(x, edge_index, z128, z16, ones)
    b_all = (b_self + 0.5 * (b_s2d + b_d2s)).reshape(1, D)
    return _tc_combine(x, sums, degs, W_self.T, W_s2d.T, W_d2s.T, b_all)


# speedup vs baseline: 3.6473x; 3.6473x over previous
import functools

import jax
import jax.numpy as jnp
from jax import lax
from jax.experimental import pallas as pl
from jax.experimental.pallas import tpu as pltpu
from jax.experimental.pallas import tpu_sc as plsc

N = 10000
E = 320000
D = 128
CHUNK = 128
NT = 16
DEGW = 16
N_PAD = 10240
TROWS = N_PAD // NT
TCH = 128
TNCH = TROWS // TCH
EC_PAD = 327680
TILE_CHUNKS = EC_PAD // (NT * CHUNK)


def _build_sc_aggregate():
    mesh = plsc.VectorSubcoreMesh(core_axis_name="c", subcore_axis_name="s")

    @functools.partial(
        pl.kernel,
        out_type=[
            jax.ShapeDtypeStruct((2, N_PAD, D), jnp.float32),
            jax.ShapeDtypeStruct((2, N_PAD, DEGW), jnp.float32),
        ],
        mesh=mesh,
        scratch_types=[
            pltpu.VMEM((CHUNK,), jnp.int32),
            pltpu.VMEM((1, CHUNK), jnp.int32),
            pltpu.VMEM((CHUNK, D), jnp.float32),
            pltpu.VMEM((CHUNK, DEGW), jnp.float32),
            pltpu.VMEM_SHARED((N_PAD, D), jnp.float32),
            pltpu.VMEM_SHARED((N_PAD, DEGW), jnp.float32),
            pltpu.SemaphoreType.DMA,
        ],
        compiler_params=pltpu.CompilerParams(use_tc_tiling_on_sc=False),
    )
    def sc_agg(x_hbm, gath_hbm, scat_hbm, z128_hbm, z16_hbm, ones_hbm,
               sums_out, deg_out,
               idx_g, idx_s, rows, ones_v, acc, dacc, sem):
        cid = lax.axis_index("c")
        tid = lax.axis_index("s")
        ebase = cid * EC_PAD

        pltpu.sync_copy(z128_hbm, rows)
        pltpu.sync_copy(z16_hbm, ones_v)
        r0 = tid * TROWS

        def zbody(k, carry):
            pltpu.sync_copy(rows, acc.at[pl.ds(r0 + k * TCH, TCH)])
            pltpu.sync_copy(ones_v, dacc.at[pl.ds(r0 + k * TCH, TCH)])
            return carry

        lax.fori_loop(0, TNCH, zbody, 0)
        pltpu.sync_copy(ones_hbm, ones_v)
        plsc.subcore_barrier()

        def body(c, carry):
            off = ebase + (tid * TILE_CHUNKS + c) * CHUNK
            pltpu.sync_copy(gath_hbm.at[pl.ds(off, CHUNK)], idx_g)
            pltpu.sync_copy(scat_hbm.at[pl.ds(off, CHUNK)], idx_s.at[0])
            pltpu.async_copy(x_hbm.at[idx_g], rows, sem).wait()
            pltpu.sync_copy(rows, acc.at[idx_s.at[0]], add=True)
            pltpu.sync_copy(ones_v, dacc.at[idx_s.at[0]], add=True)
            return carry

        lax.fori_loop(0, TILE_CHUNKS, body, 0)
        plsc.subcore_barrier()

        def wbody(k, carry):
            rr = r0 + k * TCH
            pltpu.sync_copy(acc.at[pl.ds(rr, TCH)], rows)
            pltpu.sync_copy(rows, sums_out.at[cid, pl.ds(rr, TCH)])
            pltpu.sync_copy(dacc.at[pl.ds(rr, TCH)], ones_v)
            pltpu.sync_copy(ones_v, deg_out.at[cid, pl.ds(rr, TCH)])
            return carry

        lax.fori_loop(0, TNCH, wbody, 0)

    return sc_agg


_SC_AGG = _build_sc_aggregate()

_TC_ROWS = 1000


def _tc_body(x_ref, s_ref, d_ref, ws_ref, w2_ref, w3_ref, b_ref, o_ref):
    xb = x_ref[...]
    s2d = s_ref[0]
    d2s = s_ref[1]
    r_in = 0.5 / jnp.maximum(d_ref[0, :, 0:1], 1.0)
    r_out = 0.5 / jnp.maximum(d_ref[1, :, 0:1], 1.0)
    acc = jnp.dot(xb, ws_ref[...], preferred_element_type=jnp.float32)
    acc = acc + r_in * jnp.dot(s2d, w2_ref[...], preferred_element_type=jnp.float32)
    acc = acc + r_out * jnp.dot(d2s, w3_ref[...], preferred_element_type=jnp.float32)
    o_ref[...] = acc + b_ref[...]


def _tc_combine(x, sums, degs, ws_t, w2_t, w3_t, b_all):
    return pl.pallas_call(
        _tc_body,
        grid=(N // _TC_ROWS,),
        in_specs=[
            pl.BlockSpec((_TC_ROWS, D), lambda i: (i, 0)),
            pl.BlockSpec((2, _TC_ROWS, D), lambda i: (0, i, 0)),
            pl.BlockSpec((2, _TC_ROWS, DEGW), lambda i: (0, i, 0)),
            pl.BlockSpec((D, D), lambda i: (0, 0)),
            pl.BlockSpec((D, D), lambda i: (0, 0)),
            pl.BlockSpec((D, D), lambda i: (0, 0)),
            pl.BlockSpec((1, D), lambda i: (0, 0)),
        ],
        out_specs=pl.BlockSpec((_TC_ROWS, D), lambda i: (i, 0)),
        out_shape=jax.ShapeDtypeStruct((N, D), jnp.float32),
    )(x, sums, degs, ws_t, w2_t, w3_t, b_all)


def kernel(x, edge_index, W_self, b_self, W_s2d, b_s2d, W_d2s, b_d2s):
    z128 = jnp.zeros((TCH, D), jnp.float32)
    z16 = jnp.zeros((CHUNK, DEGW), jnp.float32)
    ones = jnp.ones((CHUNK, DEGW), jnp.float32)
    src = edge_index[0]
    dst = edge_index[1]
    padg = jnp.zeros((EC_PAD - E,), jnp.int32)
    pads = jnp.full((EC_PAD - E,), N, jnp.int32)
    gath = jnp.concatenate([src, padg, dst, padg])
    scat = jnp.concatenate([dst, pads, src, pads])
    sums, degs = _SC_AGG(x, gath, scat, z128, z16, ones)
    b_all = (b_self + 0.5 * (b_s2d + b_d2s)).reshape(1, D)
    return _tc_combine(x, sums, degs, W_self.T, W_s2d.T, W_d2s.T, b_all)


# speedup vs baseline: 4.3190x; 1.1842x over previous
import functools

import jax
import jax.numpy as jnp
from jax import lax
from jax.experimental import pallas as pl
from jax.experimental.pallas import tpu as pltpu
from jax.experimental.pallas import tpu_sc as plsc

N = 10000
E = 320000
D = 128
CHUNK = 128
NT = 16
DEGW = 16
N_PAD = 10240
TROWS = N_PAD // NT
TCH = 128
TNCH = TROWS // TCH
EC_PAD = 327680
TILE_CHUNKS = EC_PAD // (NT * CHUNK)
SG = 8
GROUPS = TILE_CHUNKS // SG


def _build_sc_aggregate():
    mesh = plsc.VectorSubcoreMesh(core_axis_name="c", subcore_axis_name="s")

    @functools.partial(
        pl.kernel,
        out_type=[
            jax.ShapeDtypeStruct((2, N_PAD, D), jnp.float32),
            jax.ShapeDtypeStruct((2, N_PAD, DEGW), jnp.float32),
        ],
        mesh=mesh,
        scratch_types=[
            pltpu.VMEM((SG * CHUNK,), jnp.int32),
            pltpu.VMEM((SG, CHUNK), jnp.int32),
            pltpu.VMEM((CHUNK, D), jnp.float32),
            pltpu.VMEM((CHUNK, D), jnp.float32),
            pltpu.VMEM((CHUNK, DEGW), jnp.float32),
            pltpu.VMEM_SHARED((N_PAD, D), jnp.float32),
            pltpu.VMEM_SHARED((N_PAD, DEGW), jnp.float32),
            pltpu.SemaphoreType.DMA,
            pltpu.SemaphoreType.DMA,
            pltpu.SemaphoreType.DMA,
        ],
        compiler_params=pltpu.CompilerParams(use_tc_tiling_on_sc=False),
    )
    def sc_agg(x_hbm, gath_hbm, scat_hbm, z128_hbm, z16_hbm, ones_hbm,
               sums_out, deg_out,
               idx_g, idx_s, rows0, rows1, ones_v,
               acc, dacc, gsem, ssem, dsem):
        cid = lax.axis_index("c")
        tid = lax.axis_index("s")

        pltpu.sync_copy(z128_hbm, rows0)
        pltpu.sync_copy(z16_hbm, ones_v)
        r0 = tid * TROWS

        def zbody(k, carry):
            pltpu.sync_copy(rows0, acc.at[pl.ds(r0 + k * TCH, TCH)])
            pltpu.sync_copy(ones_v, dacc.at[pl.ds(r0 + k * TCH, TCH)])
            return carry

        lax.fori_loop(0, TNCH, zbody, 0)
        pltpu.sync_copy(ones_hbm, ones_v)

        plsc.subcore_barrier()

        goff = cid * EC_PAD + tid * TILE_CHUNKS * CHUNK
        crow = cid * NT * TILE_CHUNKS + tid * TILE_CHUNKS
        slots = (rows0, rows1)

        def body(t, carry):
            pltpu.sync_copy(gath_hbm.at[pl.ds(goff + t * SG * CHUNK, SG * CHUNK)], idx_g)
            pltpu.sync_copy(scat_hbm.at[pl.ds(crow + t * SG, SG)], idx_s)
            gd = [None, None]
            sd = [None, None]
            dd = [None, None]
            gd[0] = pltpu.async_copy(
                x_hbm.at[idx_g.at[pl.ds(0, CHUNK)]], slots[0], gsem)
            for u in range(SG):
                s = u % 2
                if u + 1 < SG:
                    if sd[(u + 1) % 2] is not None:
                        sd[(u + 1) % 2].wait()
                        dd[(u + 1) % 2].wait()
                    gd[(u + 1) % 2] = pltpu.async_copy(
                        x_hbm.at[idx_g.at[pl.ds((u + 1) * CHUNK, CHUNK)]],
                        slots[(u + 1) % 2], gsem)
                gd[s].wait()
                sd[s] = pltpu.async_copy(
                    slots[s], acc.at[idx_s.at[u]], ssem, add=True)
                dd[s] = pltpu.async_copy(
                    ones_v, dacc.at[idx_s.at[u]], dsem, add=True)
            for s in range(2):
                sd[s].wait()
                dd[s].wait()
            return carry

        lax.fori_loop(0, GROUPS, body, 0)
        plsc.subcore_barrier()

        def wbody(k, carry):
            rr = r0 + k * TCH
            pltpu.sync_copy(acc.at[pl.ds(rr, TCH)], rows0)
            pltpu.sync_copy(rows0, sums_out.at[cid, pl.ds(rr, TCH)])
            pltpu.sync_copy(dacc.at[pl.ds(rr, TCH)], ones_v)
            pltpu.sync_copy(ones_v, deg_out.at[cid, pl.ds(rr, TCH)])
            return carry

        lax.fori_loop(0, TNCH, wbody, 0)

    return sc_agg


_SC_AGG = _build_sc_aggregate()

_TC_ROWS = 1000


def _tc_body(x_ref, s_ref, d_ref, ws_ref, w2_ref, w3_ref, b_ref, o_ref):
    xb = x_ref[...]
    s2d = s_ref[0]
    d2s = s_ref[1]
    r_in = 0.5 / jnp.maximum(d_ref[0, :, 0:1], 1.0)
    r_out = 0.5 / jnp.maximum(d_ref[1, :, 0:1], 1.0)
    acc = jnp.dot(xb, ws_ref[...], preferred_element_type=jnp.float32)
    acc = acc + r_in * jnp.dot(s2d, w2_ref[...], preferred_element_type=jnp.float32)
    acc = acc + r_out * jnp.dot(d2s, w3_ref[...], preferred_element_type=jnp.float32)
    o_ref[...] = acc + b_ref[...]


def _tc_combine(x, sums, degs, ws_t, w2_t, w3_t, b_all):
    return pl.pallas_call(
        _tc_body,
        grid=(N // _TC_ROWS,),
        in_specs=[
            pl.BlockSpec((_TC_ROWS, D), lambda i: (i, 0)),
            pl.BlockSpec((2, _TC_ROWS, D), lambda i: (0, i, 0)),
            pl.BlockSpec((2, _TC_ROWS, DEGW), lambda i: (0, i, 0)),
            pl.BlockSpec((D, D), lambda i: (0, 0)),
            pl.BlockSpec((D, D), lambda i: (0, 0)),
            pl.BlockSpec((D, D), lambda i: (0, 0)),
            pl.BlockSpec((1, D), lambda i: (0, 0)),
        ],
        out_specs=pl.BlockSpec((_TC_ROWS, D), lambda i: (i, 0)),
        out_shape=jax.ShapeDtypeStruct((N, D), jnp.float32),
    )(x, sums, degs, ws_t, w2_t, w3_t, b_all)


def kernel(x, edge_index, W_self, b_self, W_s2d, b_s2d, W_d2s, b_d2s):
    z128 = jnp.zeros((TCH, D), jnp.float32)
    z16 = jnp.zeros((CHUNK, DEGW), jnp.float32)
    ones = jnp.ones((CHUNK, DEGW), jnp.float32)
    src = edge_index[0]
    dst = edge_index[1]
    padg = jnp.zeros((EC_PAD - E,), jnp.int32)
    pads = jnp.full((EC_PAD - E,), N, jnp.int32)
    gath = jnp.concatenate([src, padg, dst, padg])
    scat = jnp.concatenate([dst, pads, src, pads]).reshape(-1, CHUNK)
    sums, degs = _SC_AGG(x, gath, scat, z128, z16, ones)
    b_all = (b_self + 0.5 * (b_s2d + b_d2s)).reshape(1, D)
    return _tc_combine(x, sums, degs, W_self.T, W_s2d.T, W_d2s.T, b_all)


# speedup vs baseline: 7.9778x; 1.8471x over previous
import functools

import jax
import jax.numpy as jnp
from jax import lax
from jax.experimental import pallas as pl
from jax.experimental.pallas import tpu as pltpu
from jax.experimental.pallas import tpu_sc as plsc

N = 10000
E = 320000
D = 128
DH = 64
CHUNK = 128
NT = 16
DEGW = 16
N_PAD = 10240
TROWS = N_PAD // NT
TCH = 128
TNCH = TROWS // TCH
EC_PAD = 327680
TILE_CHUNKS = EC_PAD // (NT * CHUNK)
SG = 8
GROUPS = TILE_CHUNKS // SG
NSLOT = 4


def _build_sc_aggregate():
    mesh = plsc.VectorSubcoreMesh(core_axis_name="c", subcore_axis_name="s")

    @functools.partial(
        pl.kernel,
        out_type=[
            jax.ShapeDtypeStruct((2, N_PAD, DH), jnp.float32),
            jax.ShapeDtypeStruct((2, N_PAD, DEGW), jnp.float32),
        ],
        mesh=mesh,
        scratch_types=[
            pltpu.VMEM((SG * CHUNK,), jnp.int32),
            pltpu.VMEM((SG, CHUNK), jnp.int32),
            pltpu.VMEM((CHUNK, DH), jnp.float32),
            pltpu.VMEM((CHUNK, DH), jnp.float32),
            pltpu.VMEM((CHUNK, DH), jnp.float32),
            pltpu.VMEM((CHUNK, DH), jnp.float32),
            pltpu.VMEM((CHUNK, DEGW), jnp.float32),
            pltpu.VMEM_SHARED((N_PAD, DH), jnp.float32),
            pltpu.VMEM_SHARED((N_PAD, DH), jnp.float32),
            pltpu.VMEM_SHARED((N_PAD, DEGW), jnp.float32),
            pltpu.SemaphoreType.DMA,
            pltpu.SemaphoreType.DMA,
            pltpu.SemaphoreType.DMA,
        ],
        compiler_params=pltpu.CompilerParams(use_tc_tiling_on_sc=False),
    )
    def sc_agg(x_hbm, gath_hbm, scat_hbm, z64_hbm, z16_hbm, ones_hbm,
               sums_out, deg_out,
               idx_g, idx_s, rows0, rows1, rows2, rows3, ones_v,
               x_sp, acc, dacc, gsem, ssem, dsem):
        cid = lax.axis_index("c")
        tid = lax.axis_index("s")

        pltpu.sync_copy(z64_hbm, rows0)
        pltpu.sync_copy(z16_hbm, ones_v)
        r0 = tid * TROWS

        def zbody(k, carry):
            pltpu.sync_copy(rows0, acc.at[pl.ds(r0 + k * TCH, TCH)])
            pltpu.sync_copy(ones_v, dacc.at[pl.ds(r0 + k * TCH, TCH)])
            return carry

        lax.fori_loop(0, TNCH, zbody, 0)
        pltpu.sync_copy(ones_hbm, ones_v)

        def xbody(k, carry):
            rr = (tid + k * NT) * CHUNK

            @pl.when(rr + CHUNK <= N)
            def _():
                pltpu.sync_copy(x_hbm.at[pl.ds(rr, CHUNK)], rows1)
                pltpu.sync_copy(rows1, x_sp.at[pl.ds(rr, CHUNK)])

            @pl.when((rr < N) & (rr + CHUNK > N))
            def _():
                pltpu.sync_copy(x_hbm.at[pl.ds(rr, N - (N // CHUNK) * CHUNK)],
                                rows1.at[pl.ds(0, N - (N // CHUNK) * CHUNK)])
                pltpu.sync_copy(rows1.at[pl.ds(0, N - (N // CHUNK) * CHUNK)],
                                x_sp.at[pl.ds(rr, N - (N // CHUNK) * CHUNK)])
            return carry

        lax.fori_loop(0, (N + NT * CHUNK - 1) // (NT * CHUNK), xbody, 0)
        plsc.subcore_barrier()

        goff = cid * EC_PAD + tid * TILE_CHUNKS * CHUNK
        crow = cid * NT * TILE_CHUNKS + tid * TILE_CHUNKS
        slots = (rows0, rows1, rows2, rows3)

        def body(t, carry):
            pltpu.sync_copy(gath_hbm.at[pl.ds(goff + t * SG * CHUNK, SG * CHUNK)], idx_g)
            pltpu.sync_copy(scat_hbm.at[pl.ds(crow + t * SG, SG)], idx_s)

            def gather(u):
                return pltpu.async_copy(
                    x_sp.at[idx_g.at[pl.ds(u * CHUNK, CHUNK)]],
                    slots[u % NSLOT], gsem)

            gd = {}
            sd = {}
            dd = {}
            for p in range(NSLOT - 1):
                gd[p] = gather(p)
            for u in range(SG):
                s = u % NSLOT
                f = u + NSLOT - 1
                if f < SG:
                    fs = f % NSLOT
                    if fs in sd:
                        sd[fs].wait()
                        dd[fs].wait()
                    gd[f] = gather(f)
                gd[u].wait()
                sd[s] = pltpu.async_copy(
                    slots[s], acc.at[idx_s.at[u]], ssem, add=True)
                dd[s] = pltpu.async_copy(
                    ones_v, dacc.at[idx_s.at[u]], dsem, add=True)
            for s in range(NSLOT):
                sd[s].wait()
                dd[s].wait()
            return carry

        lax.fori_loop(0, GROUPS, body, 0)
        plsc.subcore_barrier()

        def wbody(k, carry):
            rr = r0 + k * TCH
            pltpu.sync_copy(acc.at[pl.ds(rr, TCH)], rows0)
            pltpu.sync_copy(rows0, sums_out.at[cid, pl.ds(rr, TCH)])
            pltpu.sync_copy(dacc.at[pl.ds(rr, TCH)], ones_v)
            pltpu.sync_copy(ones_v, deg_out.at[cid, pl.ds(rr, TCH)])
            return carry

        lax.fori_loop(0, TNCH, wbody, 0)

    return sc_agg


_SC_AGG = _build_sc_aggregate()

_TC_ROWS = 1000


def _tc_body(x_ref, s0_ref, s1_ref, d_ref, ws_ref, w2_ref, w3_ref, b_ref, o_ref):
    xb = x_ref[...]
    r_in = 0.5 / jnp.maximum(d_ref[0, :, 0:1], 1.0)
    r_out = 0.5 / jnp.maximum(d_ref[1, :, 0:1], 1.0)
    acc = jnp.dot(xb, ws_ref[...], preferred_element_type=jnp.float32)
    s2d = (jnp.dot(s0_ref[0], w2_ref[pl.ds(0, DH), :], preferred_element_type=jnp.float32)
           + jnp.dot(s1_ref[0], w2_ref[pl.ds(DH, DH), :], preferred_element_type=jnp.float32))
    d2s = (jnp.dot(s0_ref[1], w3_ref[pl.ds(0, DH), :], preferred_element_type=jnp.float32)
           + jnp.dot(s1_ref[1], w3_ref[pl.ds(DH, DH), :], preferred_element_type=jnp.float32))
    o_ref[...] = acc + r_in * s2d + r_out * d2s + b_ref[...]


def _tc_combine(x, sums0, sums1, degs, ws_t, w2_t, w3_t, b_all):
    return pl.pallas_call(
        _tc_body,
        grid=(N // _TC_ROWS,),
        in_specs=[
            pl.BlockSpec((_TC_ROWS, D), lambda i: (i, 0)),
            pl.BlockSpec((2, _TC_ROWS, DH), lambda i: (0, i, 0)),
            pl.BlockSpec((2, _TC_ROWS, DH), lambda i: (0, i, 0)),
            pl.BlockSpec((2, _TC_ROWS, DEGW), lambda i: (0, i, 0)),
            pl.BlockSpec((D, D), lambda i: (0, 0)),
            pl.BlockSpec((D, D), lambda i: (0, 0)),
            pl.BlockSpec((D, D), lambda i: (0, 0)),
            pl.BlockSpec((1, D), lambda i: (0, 0)),
        ],
        out_specs=pl.BlockSpec((_TC_ROWS, D), lambda i: (i, 0)),
        out_shape=jax.ShapeDtypeStruct((N, D), jnp.float32),
    )(x, sums0, sums1, degs, ws_t, w2_t, w3_t, b_all)


def kernel(x, edge_index, W_self, b_self, W_s2d, b_s2d, W_d2s, b_d2s):
    z64 = jnp.zeros((TCH, DH), jnp.float32)
    z16 = jnp.zeros((CHUNK, DEGW), jnp.float32)
    ones = jnp.ones((CHUNK, DEGW), jnp.float32)
    src = edge_index[0]
    dst = edge_index[1]
    padg = jnp.zeros((EC_PAD - E,), jnp.int32)
    pads = jnp.full((EC_PAD - E,), N, jnp.int32)
    gath = jnp.concatenate([src, padg, dst, padg])
    scat = jnp.concatenate([dst, pads, src, pads]).reshape(-1, CHUNK)
    sums0, degs = _SC_AGG(x[:, :DH], gath, scat, z64, z16, ones)
    sums1, _ = _SC_AGG(x[:, DH:], gath, scat, z64, z16, ones)
    b_all = (b_self + 0.5 * (b_s2d + b_d2s)).reshape(1, D)
    return _tc_combine(x, sums0, sums1, degs, W_self.T, W_s2d.T, W_d2s.T, b_all)


# speedup vs baseline: 8.4573x; 1.0601x over previous
import functools

import jax
import jax.numpy as jnp
from jax import lax
from jax.experimental import pallas as pl
from jax.experimental.pallas import tpu as pltpu
from jax.experimental.pallas import tpu_sc as plsc

N = 10000
E = 320000
D = 128
DH = 64
CHUNK = 128
NT = 16
DEGW = 16
N_PAD = 10240
TROWS = N_PAD // NT
TCH = 128
TNCH = TROWS // TCH
EC_PAD = 327680
TILE_CHUNKS = EC_PAD // (NT * CHUNK)
SG = 8
GROUPS = TILE_CHUNKS // SG
NSLOT = 4


def _build_sc_aggregate(with_deg: bool):
    mesh = plsc.VectorSubcoreMesh(core_axis_name="c", subcore_axis_name="s")
    out_type = [jax.ShapeDtypeStruct((2, N_PAD, DH), jnp.float32)]
    if with_deg:
        out_type.append(jax.ShapeDtypeStruct((2, N_PAD, DEGW), jnp.float32))

    @functools.partial(
        pl.kernel,
        out_type=out_type,
        mesh=mesh,
        scratch_types=[
            pltpu.VMEM((SG * CHUNK,), jnp.int32),
            pltpu.VMEM((SG, CHUNK), jnp.int32),
            pltpu.VMEM((CHUNK, DH), jnp.float32),
            pltpu.VMEM((CHUNK, DH), jnp.float32),
            pltpu.VMEM((CHUNK, DH), jnp.float32),
            pltpu.VMEM((CHUNK, DH), jnp.float32),
            pltpu.VMEM((CHUNK, DEGW), jnp.float32),
            pltpu.VMEM_SHARED((N_PAD, DH), jnp.float32),
            pltpu.VMEM_SHARED((N_PAD, DH), jnp.float32),
            pltpu.VMEM_SHARED((N_PAD, DEGW), jnp.float32),
            pltpu.SemaphoreType.DMA,
            pltpu.SemaphoreType.DMA,
            pltpu.SemaphoreType.DMA,
        ],
        compiler_params=pltpu.CompilerParams(use_tc_tiling_on_sc=False),
    )
    def sc_agg(x_hbm, gath_hbm, scat_hbm, z64_hbm, z16_hbm, ones_hbm,
               sums_out, *rest):
        if with_deg:
            (deg_out, idx_g, idx_s, rows0, rows1, rows2, rows3, ones_v,
             x_sp, acc, dacc, gsem, ssem, dsem) = rest
        else:
            (idx_g, idx_s, rows0, rows1, rows2, rows3, ones_v,
             x_sp, acc, dacc, gsem, ssem, dsem) = rest
        cid = lax.axis_index("c")
        tid = lax.axis_index("s")

        pltpu.sync_copy(z64_hbm, rows0)
        pltpu.sync_copy(z16_hbm, ones_v)
        r0 = tid * TROWS

        def zbody(k, carry):
            pltpu.sync_copy(rows0, acc.at[pl.ds(r0 + k * TCH, TCH)])
            if with_deg:
                pltpu.sync_copy(ones_v, dacc.at[pl.ds(r0 + k * TCH, TCH)])
            return carry

        lax.fori_loop(0, TNCH, zbody, 0)
        pltpu.sync_copy(ones_hbm, ones_v)

        def xbody(k, carry):
            rr = (tid + k * NT) * CHUNK

            @pl.when(rr + CHUNK <= N)
            def _():
                pltpu.sync_copy(x_hbm.at[pl.ds(rr, CHUNK)], rows1)
                pltpu.sync_copy(rows1, x_sp.at[pl.ds(rr, CHUNK)])

            @pl.when((rr < N) & (rr + CHUNK > N))
            def _():
                pltpu.sync_copy(x_hbm.at[pl.ds(rr, N - (N // CHUNK) * CHUNK)],
                                rows1.at[pl.ds(0, N - (N // CHUNK) * CHUNK)])
                pltpu.sync_copy(rows1.at[pl.ds(0, N - (N // CHUNK) * CHUNK)],
                                x_sp.at[pl.ds(rr, N - (N // CHUNK) * CHUNK)])
            return carry

        lax.fori_loop(0, (N + NT * CHUNK - 1) // (NT * CHUNK), xbody, 0)
        plsc.subcore_barrier()

        goff = cid * EC_PAD + tid * TILE_CHUNKS * CHUNK
        crow = cid * NT * TILE_CHUNKS + tid * TILE_CHUNKS
        slots = (rows0, rows1, rows2, rows3)

        def body(t, carry):
            pltpu.sync_copy(gath_hbm.at[pl.ds(goff + t * SG * CHUNK, SG * CHUNK)], idx_g)
            pltpu.sync_copy(scat_hbm.at[pl.ds(crow + t * SG, SG)], idx_s)

            def gather(u):
                return pltpu.async_copy(
                    x_sp.at[idx_g.at[pl.ds(u * CHUNK, CHUNK)]],
                    slots[u % NSLOT], gsem)

            gd = {}
            sd = {}
            dd = {}
            for p in range(NSLOT - 1):
                gd[p] = gather(p)
            for u in range(SG):
                s = u % NSLOT
                f = u + NSLOT - 1
                if f < SG:
                    fs = f % NSLOT
                    if fs in sd:
                        sd[fs].wait()
                        if with_deg:
                            dd[fs].wait()
                    gd[f] = gather(f)
                gd[u].wait()
                sd[s] = pltpu.async_copy(
                    slots[s], acc.at[idx_s.at[u]], ssem, add=True)
                if with_deg:
                    dd[s] = pltpu.async_copy(
                        ones_v, dacc.at[idx_s.at[u]], dsem, add=True)
            for s in range(NSLOT):
                sd[s].wait()
                if with_deg:
                    dd[s].wait()
            return carry

        lax.fori_loop(0, GROUPS, body, 0)
        plsc.subcore_barrier()

        def wbody(k, carry):
            rr = r0 + k * TCH
            pltpu.sync_copy(acc.at[pl.ds(rr, TCH)], rows0)
            pltpu.sync_copy(rows0, sums_out.at[cid, pl.ds(rr, TCH)])
            if with_deg:
                pltpu.sync_copy(dacc.at[pl.ds(rr, TCH)], ones_v)
                pltpu.sync_copy(ones_v, deg_out.at[cid, pl.ds(rr, TCH)])
            return carry

        lax.fori_loop(0, TNCH, wbody, 0)

    return sc_agg


_SC_AGG_DEG = _build_sc_aggregate(True)
_SC_AGG = _build_sc_aggregate(False)

_TC_ROWS = 1000


def _tc_body(x_ref, s0_ref, s1_ref, d_ref, ws_ref, w2_ref, w3_ref, b_ref, o_ref):
    xb = x_ref[...]
    r_in = 0.5 / jnp.maximum(d_ref[0, :, 0:1], 1.0)
    r_out = 0.5 / jnp.maximum(d_ref[1, :, 0:1], 1.0)
    acc = jnp.dot(xb, ws_ref[...], preferred_element_type=jnp.float32)
    s2d = (jnp.dot(s0_ref[0], w2_ref[pl.ds(0, DH), :], preferred_element_type=jnp.float32)
           + jnp.dot(s1_ref[0], w2_ref[pl.ds(DH, DH), :], preferred_element_type=jnp.float32))
    d2s = (jnp.dot(s0_ref[1], w3_ref[pl.ds(0, DH), :], preferred_element_type=jnp.float32)
           + jnp.dot(s1_ref[1], w3_ref[pl.ds(DH, DH), :], preferred_element_type=jnp.float32))
    o_ref[...] = acc + r_in * s2d + r_out * d2s + b_ref[...]


def _tc_combine(x, sums0, sums1, degs, ws_t, w2_t, w3_t, b_all):
    return pl.pallas_call(
        _tc_body,
        grid=(N // _TC_ROWS,),
        in_specs=[
            pl.BlockSpec((_TC_ROWS, D), lambda i: (i, 0)),
            pl.BlockSpec((2, _TC_ROWS, DH), lambda i: (0, i, 0)),
            pl.BlockSpec((2, _TC_ROWS, DH), lambda i: (0, i, 0)),
            pl.BlockSpec((2, _TC_ROWS, DEGW), lambda i: (0, i, 0)),
            pl.BlockSpec((D, D), lambda i: (0, 0)),
            pl.BlockSpec((D, D), lambda i: (0, 0)),
            pl.BlockSpec((D, D), lambda i: (0, 0)),
            pl.BlockSpec((1, D), lambda i: (0, 0)),
        ],
        out_specs=pl.BlockSpec((_TC_ROWS, D), lambda i: (i, 0)),
        out_shape=jax.ShapeDtypeStruct((N, D), jnp.float32),
    )(x, sums0, sums1, degs, ws_t, w2_t, w3_t, b_all)


def kernel(x, edge_index, W_self, b_self, W_s2d, b_s2d, W_d2s, b_d2s):
    z64 = jnp.zeros((TCH, DH), jnp.float32)
    z16 = jnp.zeros((CHUNK, DEGW), jnp.float32)
    ones = jnp.ones((CHUNK, DEGW), jnp.float32)
    src = edge_index[0]
    dst = edge_index[1]
    padg = jnp.zeros((EC_PAD - E,), jnp.int32)
    pads = jnp.full((EC_PAD - E,), N, jnp.int32)
    gath = jnp.concatenate([src, padg, dst, padg])
    scat = jnp.concatenate([dst, pads, src, pads]).reshape(-1, CHUNK)
    sums0, degs = _SC_AGG_DEG(x[:, :DH], gath, scat, z64, z16, ones)
    (sums1,) = _SC_AGG(x[:, DH:], gath, scat, z64, z16, ones)
    b_all = (b_self + 0.5 * (b_s2d + b_d2s)).reshape(1, D)
    return _tc_combine(x, sums0, sums1, degs, W_self.T, W_s2d.T, W_d2s.T, b_all)
